# baseline (speedup 1.0000x reference)
import functools

import jax
import jax.numpy as jnp
from jax import lax
from jax.experimental import pallas as pl
from jax.experimental.pallas import tpu as pltpu
from jax.experimental.pallas import tpu_sc as plsc

K = 8
B, D, N = 4, 2048, 8192
R = B * D
L = 16
VPR = N // L
G = 8
NG = VPR // G
NC, NS = 2, 16
NW = NC * NS
RPW = R // NW
CR = 4
CW = CR * N
NCH = RPW // CR
NSLOT = VPR + 4
NEG = float("-inf")


def _sortd(v):
    sk, _ = plsc.sort_key_val(v, v, descending=True)
    return sk


def _msort(a, b, lane):
    comb = jnp.where(lane < K, a, lax.rev(b, (0,)))
    return _sortd(comb)


def _tree_max(vals):
    vals = list(vals)
    while len(vals) > 1:
        nxt = [jnp.maximum(vals[2 * i], vals[2 * i + 1])
               for i in range(len(vals) // 2)]
        if len(vals) % 2:
            nxt.append(vals[-1])
        vals = nxt
    return vals[0]


def _sc_body(x_hbm, out_hbm, buf, gmax, slots, stage, nslot, sem0, sem1):
    cid = lax.axis_index("c")
    sid = lax.axis_index("s")
    w = sid * NC + cid
    row0 = w * RPW
    base_off = row0 * N
    b_idx = w // (D // RPW)
    d0 = (w % (D // RPW)) * RPW

    lane = lax.iota(jnp.int32, L)
    lt8 = lane < K

    def copy(c, par, sem):
        return pltpu.make_async_copy(
            x_hbm.at[pl.ds(base_off + c * CW, CW)],
            buf.at[pl.ds(par * CW, CW)],
            sem,
        )

    copy(0, 0, sem0).start()
    copy(1, 1, sem1).start()

    def chunk_body(c, carry):
        par = c & 1
        pbase = par * CW

        @pl.when(par == 0)
        def _():
            copy(c, 0, sem0).wait()

        @pl.when(par == 1)
        def _():
            copy(c, 1, sem1).wait()

        def row_body(r, _):
            rb = pbase + r * N

            accs = [None] * 8
            for g in range(NG):
                gb0 = rb + g * (G * L)
                xs = [buf[pl.ds(gb0 + i * L, L)] for i in range(G)]
                gm = _tree_max(xs)
                gmax[pl.ds(g * L, L)] = gm
                a = g % 8
                accs[a] = gm if accs[a] is None else jnp.maximum(accs[a], gm)
            m_run = _tree_max(accs)

            sm = _sortd(m_run)
            t_s = jnp.max(jnp.where(lane == K - 1, sm, NEG))
            t_vec = jnp.full((L,), t_s, jnp.float32)

            nslot[0] = 0

            def g2(g, _2):
                gm = gmax[pl.ds(g * L, L)]

                @pl.when(jnp.any(gm >= t_vec))
                def _():
                    gb0 = rb + g * (G * L)
                    for i in range(G):
                        x = buf[pl.ds(gb0 + i * L, L)]
                        mask = x >= t_vec

                        @pl.when(jnp.any(mask))
                        def _():
                            ns = nslot[0]
                            slots[pl.ds(ns * L, L)] = jnp.where(mask, x, NEG)
                            nslot[0] = ns + 1

                return 0

            lax.fori_loop(0, NG, g2, 0, unroll=2)

            n = nslot[0]
            neg_v = jnp.full((L,), NEG, jnp.float32)
            for p in range(4):
                slots[pl.ds((n + p) * L, L)] = neg_v
            nq = (n + 3) >> 2

            def tmerge(ci, acc):
                bw = ci * (4 * L)
                s0 = _sortd(slots[pl.ds(bw, L)])
                s1 = _sortd(slots[pl.ds(bw + L, L)])
                s2 = _sortd(slots[pl.ds(bw + 2 * L, L)])
                s3 = _sortd(slots[pl.ds(bw + 3 * L, L)])
                m = _msort(_msort(s0, s1, lane), _msort(s2, s3, lane), lane)
                return _msort(acc, m, lane)

            top8 = lax.fori_loop(0, nq, tmerge, neg_v)

            i_row = c * CR + r
            plsc.store_scatter(stage, [lane * RPW + i_row], top8, mask=lt8)
            return 0

        lax.fori_loop(0, CR, row_body, 0)

        c2 = c + 2

        @pl.when((c2 < NCH) & (par == 0))
        def _():
            copy(c2, 0, sem0).start()

        @pl.when((c2 < NCH) & (par == 1))
        def _():
            copy(c2, 1, sem1).start()

        return carry

    lax.fori_loop(0, NCH, chunk_body, 0)

    for j in range(K):
        pltpu.sync_copy(
            stage.at[pl.ds(j * RPW, RPW)],
            out_hbm.at[b_idx, j, pl.ds(d0, RPW)],
        )


@functools.partial(
    pl.kernel,
    out_type=jax.ShapeDtypeStruct((B, K, D), jnp.float32),
    mesh=plsc.VectorSubcoreMesh(core_axis_name="c", subcore_axis_name="s"),
    compiler_params=pltpu.CompilerParams(needs_layout_passes=False),
    scratch_types=[
        pltpu.VMEM((2 * CW,), jnp.float32),
        pltpu.VMEM((NG * L,), jnp.float32),
        pltpu.VMEM((NSLOT * L,), jnp.float32),
        pltpu.VMEM((K * RPW,), jnp.float32),
        pltpu.SMEM((1,), jnp.int32),
        pltpu.SemaphoreType.DMA,
        pltpu.SemaphoreType.DMA,
    ],
)
def _sc_topk(x_hbm, out_hbm, buf, gmax, slots, stage, nslot, sem0, sem1):
    _sc_body(x_hbm, out_hbm, buf, gmax, slots, stage, nslot, sem0, sem1)


def kernel(inputs):
    return _sc_topk(inputs.reshape(-1))
